# baseline (speedup 1.0000x reference)
import jax
import jax.numpy as jnp
from jax.experimental import pallas as pl

_TILE = 256


def _bf16_round(v):
    return v.astype(jnp.bfloat16).astype(jnp.float32)


def _chamfer_kernel(pT_ref, q_ref, o_ref):
    n = pT_ref.shape[2]
    px = pT_ref[0, 0:1, :]
    py = pT_ref[0, 1:2, :]
    pz = pT_ref[0, 2:3, :]
    pn = px * px + py * py + pz * pz
    pxb = _bf16_round(px)
    pyb = _bf16_round(py)
    pzb = _bf16_round(pz)

    def body(t, acc):
        base = t * _TILE
        qx = q_ref[0, pl.ds(base, _TILE), 0:1]
        qy = q_ref[0, pl.ds(base, _TILE), 1:2]
        qz = q_ref[0, pl.ds(base, _TILE), 2:3]
        qn = qx * qx + qy * qy + qz * qz
        ab = (-_bf16_round(qx)) * pxb + _bf16_round(qy) * pyb \
            + _bf16_round(qz) * pzb
        d2 = (qn + pn) - 2.0 * ab
        m = jnp.min(d2, axis=1)
        return acc + jnp.sum(m)

    acc = jax.lax.fori_loop(0, n // _TILE, body, jnp.float32(0.0))
    o_ref[0] = jnp.full((8, 128), acc, jnp.float32)


def kernel(xyz):
    B, N, _ = xyz.shape
    pT = jnp.swapaxes(xyz, 1, 2)
    out = pl.pallas_call(
        _chamfer_kernel,
        grid=(B,),
        in_specs=[
            pl.BlockSpec((1, 3, N), lambda b: (b, 0, 0)),
            pl.BlockSpec((1, N, 3), lambda b: (b, 0, 0)),
        ],
        out_specs=pl.BlockSpec((1, 8, 128), lambda b: (b, 0, 0)),
        out_shape=jax.ShapeDtypeStruct((B, 8, 128), jnp.float32),
    )(pT, xyz)
    return (2.0 / (B * N)) * jnp.sum(out[:, 0, 0])


# speedup vs baseline: 1.4874x; 1.4874x over previous
import jax
import jax.numpy as jnp
from jax.experimental import pallas as pl
from jax.experimental.pallas import tpu as pltpu

_TILE = 256
_K = 8


def _chamfer_kernel(p_ref, q_ref, o_ref):
    n = p_ref.shape[2]
    p = p_ref[0]
    px = p[0:1, :]
    py = p[1:2, :]
    pz = p[2:3, :]
    pn = px * px + py * py + pz * pz
    row = jax.lax.broadcasted_iota(jnp.int32, (_K, 1), 0)
    pb = jnp.where(row == 0, -p, p).astype(jnp.bfloat16)

    def body(t, acc):
        base = t * _TILE
        qtile = q_ref[0, pl.ds(base, _TILE), :]
        qx = qtile[:, 0:1]
        qy = qtile[:, 1:2]
        qz = qtile[:, 2:3]
        qn = qx * qx + qy * qy + qz * qz
        ab = jnp.dot(qtile.astype(jnp.bfloat16), pb,
                     preferred_element_type=jnp.float32)
        d2 = (qn + pn) - 2.0 * ab
        m = jnp.min(d2, axis=1)
        return acc + jnp.sum(m)

    acc = jax.lax.fori_loop(0, n // _TILE, body, jnp.float32(0.0))
    o_ref[0] = jnp.full((8, 128), acc, jnp.float32)


def kernel(xyz):
    B, N, _ = xyz.shape
    qmat = jnp.pad(xyz, ((0, 0), (0, 0), (0, _K - 3)))
    pmat = jnp.swapaxes(qmat, 1, 2)
    out = pl.pallas_call(
        _chamfer_kernel,
        grid=(B,),
        in_specs=[
            pl.BlockSpec((1, _K, N), lambda b: (b, 0, 0)),
            pl.BlockSpec((1, N, _K), lambda b: (b, 0, 0)),
        ],
        out_specs=pl.BlockSpec((1, 8, 128), lambda b: (b, 0, 0)),
        out_shape=jax.ShapeDtypeStruct((B, 8, 128), jnp.float32),
        compiler_params=pltpu.CompilerParams(
            dimension_semantics=("parallel",),
        ),
    )(pmat, qmat)
    return (2.0 / (B * N)) * jnp.sum(out[:, 0, 0])


# speedup vs baseline: 1.5358x; 1.0325x over previous
import jax
import jax.numpy as jnp
from jax.experimental import pallas as pl
from jax.experimental.pallas import tpu as pltpu

_TILE = 256
_K = 8


def _chamfer_kernel(p_ref, q_ref, o_ref):
    n = p_ref.shape[2]
    p = p_ref[0]
    px = p[0:1, :]
    py = p[1:2, :]
    pz = p[2:3, :]
    pn = px * px + py * py + pz * pz
    row = jax.lax.broadcasted_iota(jnp.int32, (_K, 1), 0)
    scale = jnp.where(row == 0, 2.0, -2.0).astype(jnp.float32)
    pb = (p * scale).astype(jnp.bfloat16)

    def body(t, acc):
        base = t * _TILE
        qb = q_ref[0, pl.ds(base, _TILE), :].astype(jnp.bfloat16)
        ab2 = jnp.dot(qb, pb, preferred_element_type=jnp.float32)
        m = jnp.min(pn + ab2, axis=1)
        return acc + jnp.sum(m)

    acc = jax.lax.fori_loop(0, n // _TILE, body, jnp.float32(0.0))
    total = acc + jnp.sum(pn)
    o_ref[0] = jnp.full((8, 128), total, jnp.float32)


def kernel(xyz):
    B, N, _ = xyz.shape
    qmat = jnp.pad(xyz, ((0, 0), (0, 0), (0, _K - 3)))
    pmat = jnp.swapaxes(qmat, 1, 2)
    out = pl.pallas_call(
        _chamfer_kernel,
        grid=(B,),
        in_specs=[
            pl.BlockSpec((1, _K, N), lambda b: (b, 0, 0)),
            pl.BlockSpec((1, N, _K), lambda b: (b, 0, 0)),
        ],
        out_specs=pl.BlockSpec((1, 8, 128), lambda b: (b, 0, 0)),
        out_shape=jax.ShapeDtypeStruct((B, 8, 128), jnp.float32),
        compiler_params=pltpu.CompilerParams(
            dimension_semantics=("parallel",),
        ),
    )(pmat, qmat)
    return (2.0 / (B * N)) * jnp.sum(out[:, 0, 0])


# speedup vs baseline: 1.9677x; 1.2813x over previous
import jax
import jax.numpy as jnp
from jax.experimental import pallas as pl
from jax.experimental.pallas import tpu as pltpu

_TILE = 256
_UNROLL = 2
_K = 8


def _chamfer_kernel(p_ref, q_ref, o_ref):
    n = p_ref.shape[2]
    p = p_ref[0]
    px = p[0:1, :]
    py = p[1:2, :]
    pz = p[2:3, :]
    pn = px * px + py * py + pz * pz
    hi = pn.astype(jnp.bfloat16)
    lo = (pn - hi.astype(jnp.float32)).astype(jnp.bfloat16)
    pb = jnp.concatenate(
        [
            (2.0 * px).astype(jnp.bfloat16),
            (-2.0 * py).astype(jnp.bfloat16),
            (-2.0 * pz).astype(jnp.bfloat16),
            hi,
            lo,
            jnp.zeros((_K - 5, n), jnp.bfloat16),
        ],
        axis=0,
    )

    def body(t, acc):
        for u in range(_UNROLL):
            base = (_UNROLL * t + u) * _TILE
            qb = q_ref[0, pl.ds(base, _TILE), :].astype(jnp.bfloat16)
            d = jnp.dot(qb, pb, preferred_element_type=jnp.float32)
            acc = acc + jnp.sum(jnp.min(d, axis=1))
        return acc

    acc = jax.lax.fori_loop(0, n // (_TILE * _UNROLL), body, jnp.float32(0.0))
    total = acc + jnp.sum(pn)
    o_ref[0] = jnp.full((8, 128), total, jnp.float32)


def kernel(xyz):
    B, N, _ = xyz.shape
    qmat = jnp.concatenate(
        [
            xyz,
            jnp.ones((B, N, 2), jnp.float32),
            jnp.zeros((B, N, _K - 5), jnp.float32),
        ],
        axis=2,
    )
    pmat = jnp.swapaxes(xyz, 1, 2)
    out = pl.pallas_call(
        _chamfer_kernel,
        grid=(B,),
        in_specs=[
            pl.BlockSpec((1, 3, N), lambda b: (b, 0, 0)),
            pl.BlockSpec((1, N, _K), lambda b: (b, 0, 0)),
        ],
        out_specs=pl.BlockSpec((1, 8, 128), lambda b: (b, 0, 0)),
        out_shape=jax.ShapeDtypeStruct((B, 8, 128), jnp.float32),
        compiler_params=pltpu.CompilerParams(
            dimension_semantics=("parallel",),
        ),
    )(pmat, qmat)
    return (2.0 / (B * N)) * jnp.sum(out[:, 0, 0])
